# Initial kernel scaffold; baseline (speedup 1.0000x reference)
import jax
import jax.numpy as jnp
from jax.experimental import pallas as pl


def kernel(x, edge_index, W1, att_src1, att_dst1, b1, W2, att_src2, att_dst2, b2):
    raise NotImplementedError("write your pallas kernel here")


# speedup vs baseline: 2.5240x; 2.5240x over previous
import functools

import jax
import jax.numpy as jnp
from jax import lax
from jax.experimental import pallas as pl
from jax.experimental.pallas import tpu as pltpu
from jax.experimental.pallas import tpu_sc as plsc

N_NODES = 10000
NPAD = 10240
D_IN = 128
H1 = 8
HID = 128
F1 = H1 * HID
F2 = 128
E_RAW = 320000
E_SL = E_RAW + N_NODES

NTILES = 32
W_EDGE = 1728
NWIN = 6
S_EDGE = W_EDGE * NWIN
EPAD = NTILES * S_EDGE

BLK = 512


def _pre1_body(x_ref, w_ref, asrc_ref, adst_ref, h_ref, as_ref, ad_ref):
    h = jnp.dot(x_ref[...], w_ref[...], preferred_element_type=jnp.float32)
    h_ref[...] = h
    h3 = h.reshape(BLK, H1, HID)
    dn = (((1,), (2,)), ((0,), (1,)))
    as_ref[...] = lax.dot_general(asrc_ref[...], h3, dn,
                                  precision=lax.Precision.HIGHEST,
                                  preferred_element_type=jnp.float32)
    ad_ref[...] = lax.dot_general(adst_ref[...], h3, dn,
                                  precision=lax.Precision.HIGHEST,
                                  preferred_element_type=jnp.float32)


def _pre1(x_pad, W1, att_src1, att_dst1):
    return pl.pallas_call(
        _pre1_body,
        grid=(NPAD // BLK,),
        in_specs=[
            pl.BlockSpec((BLK, D_IN), lambda i: (i, 0)),
            pl.BlockSpec((D_IN, F1), lambda i: (0, 0)),
            pl.BlockSpec((H1, HID), lambda i: (0, 0)),
            pl.BlockSpec((H1, HID), lambda i: (0, 0)),
        ],
        out_specs=[
            pl.BlockSpec((BLK, F1), lambda i: (i, 0)),
            pl.BlockSpec((H1, BLK), lambda i: (0, i)),
            pl.BlockSpec((H1, BLK), lambda i: (0, i)),
        ],
        out_shape=[
            jax.ShapeDtypeStruct((NPAD, F1), jnp.float32),
            jax.ShapeDtypeStruct((H1, NPAD), jnp.float32),
            jax.ShapeDtypeStruct((H1, NPAD), jnp.float32),
        ],
    )(x_pad, W1, att_src1, att_dst1)


def _mid_body(acc_ref, den_ref, b1_ref, w2_ref, as2_ref, ad2_ref,
              h2_ref, as_ref, ad_ref):
    acc = acc_ref[...].reshape(BLK, H1, HID)
    den = den_ref[...][:, :H1].reshape(BLK, H1, 1)
    h = acc / (den + 1e-30) + b1_ref[...].reshape(1, H1, HID)
    h = jnp.where(h > 0, h, jnp.exp(jnp.minimum(h, 0.0)) - 1.0)
    hf = h.reshape(BLK, F1)
    h2 = jnp.dot(hf, w2_ref[...], preferred_element_type=jnp.float32)
    h2_ref[...] = h2
    h23 = h2.reshape(BLK, 1, F2)
    dn = (((1,), (2,)), ((0,), (1,)))
    as_ref[...] = lax.dot_general(as2_ref[...], h23, dn,
                                  precision=lax.Precision.HIGHEST,
                                  preferred_element_type=jnp.float32)
    ad_ref[...] = lax.dot_general(ad2_ref[...], h23, dn,
                                  precision=lax.Precision.HIGHEST,
                                  preferred_element_type=jnp.float32)


def _mid(acc1, dacc1, b1_2d, W2, att_src2, att_dst2):
    return pl.pallas_call(
        _mid_body,
        grid=(NPAD // BLK,),
        in_specs=[
            pl.BlockSpec((BLK, F1), lambda i: (i, 0)),
            pl.BlockSpec((BLK, 16), lambda i: (i, 0)),
            pl.BlockSpec((1, F1), lambda i: (0, 0)),
            pl.BlockSpec((F1, F2), lambda i: (0, 0)),
            pl.BlockSpec((1, F2), lambda i: (0, 0)),
            pl.BlockSpec((1, F2), lambda i: (0, 0)),
        ],
        out_specs=[
            pl.BlockSpec((BLK, F2), lambda i: (i, 0)),
            pl.BlockSpec((1, BLK), lambda i: (0, i)),
            pl.BlockSpec((1, BLK), lambda i: (0, i)),
        ],
        out_shape=[
            jax.ShapeDtypeStruct((NPAD, F2), jnp.float32),
            jax.ShapeDtypeStruct((1, NPAD), jnp.float32),
            jax.ShapeDtypeStruct((1, NPAD), jnp.float32),
        ],
    )(acc1, dacc1, b1_2d, W2, att_src2, att_dst2)


def _post_body(acc_ref, den_ref, b2_ref, out_ref):
    o = acc_ref[...] / (den_ref[...][:, :1] + 1e-30) + b2_ref[...]
    m = jnp.max(o, axis=1, keepdims=True)
    z = o - m
    lse = jnp.log(jnp.sum(jnp.exp(z), axis=1, keepdims=True))
    out_ref[...] = z - lse


def _post(acc2, dacc2, b2_2d):
    return pl.pallas_call(
        _post_body,
        grid=(NPAD // BLK,),
        in_specs=[
            pl.BlockSpec((BLK, F2), lambda i: (i, 0)),
            pl.BlockSpec((BLK, 16), lambda i: (i, 0)),
            pl.BlockSpec((1, F2), lambda i: (0, 0)),
        ],
        out_specs=pl.BlockSpec((BLK, F2), lambda i: (i, 0)),
        out_shape=jax.ShapeDtypeStruct((NPAD, F2), jnp.float32),
    )(acc2, dacc2, b2_2d)


_MESH = plsc.VectorSubcoreMesh(core_axis_name="c", subcore_axis_name="s")
_SC_PARAMS = pltpu.CompilerParams(use_tc_tiling_on_sc=False,
                                  needs_layout_passes=False)


def _make_edge_e(Hh):
    @functools.partial(
        pl.kernel,
        out_type=jax.ShapeDtypeStruct((Hh, EPAD), jnp.float32),
        mesh=_MESH,
        compiler_params=_SC_PARAMS,
        scratch_types=[
            pltpu.VMEM((NPAD,), jnp.float32),
            pltpu.VMEM((NPAD,), jnp.float32),
            pltpu.VMEM((W_EDGE,), jnp.int32),
            pltpu.VMEM((W_EDGE,), jnp.int32),
            pltpu.VMEM((W_EDGE,), jnp.float32),
            pltpu.VMEM((16,), jnp.float32),
        ],
    )
    def edge_e(src_hbm, dst_hbm, as_hbm, ad_hbm, m_hbm, e_hbm,
               as_t, ad_t, src_w, dst_w, e_w, m_v):
        wid = lax.axis_index("s") * 2 + lax.axis_index("c")
        base = wid * S_EDGE
        pltpu.sync_copy(m_hbm, m_v)
        mv = m_v[...]
        for h in range(Hh):
            pltpu.sync_copy(as_hbm.at[h], as_t)
            pltpu.sync_copy(ad_hbm.at[h], ad_t)

            def win_body(w, carry):
                off = base + w * W_EDGE
                pltpu.sync_copy(src_hbm.at[pl.ds(off, W_EDGE)], src_w)
                pltpu.sync_copy(dst_hbm.at[pl.ds(off, W_EDGE)], dst_w)

                def grp(g, c2):
                    s16 = src_w[pl.ds(g * 16, 16)]
                    d16 = dst_w[pl.ds(g * 16, 16)]
                    a = plsc.load_gather(as_t, [s16]) \
                        + plsc.load_gather(ad_t, [d16])
                    a = jnp.where(a > 0, a, 0.2 * a)
                    e_w[pl.ds(g * 16, 16)] = jnp.exp(a - mv)
                    return c2

                lax.fori_loop(0, W_EDGE // 16, grp, 0)
                pltpu.sync_copy(e_w, e_hbm.at[h, pl.ds(off, W_EDGE)])
                return carry

            lax.fori_loop(0, NWIN, win_body, 0)

    return edge_e


def _make_seg_agg(Hh, F, CHUNK, NCHUNK):
    NPASS = NCHUNK // 2
    RPT = CHUNK // 16

    @functools.partial(
        pl.kernel,
        out_type=(
            jax.ShapeDtypeStruct((NPAD, F), jnp.float32),
            jax.ShapeDtypeStruct((NPAD, 16), jnp.float32),
        ),
        mesh=_MESH,
        compiler_params=_SC_PARAMS,
        scratch_types=[
            pltpu.VMEM((W_EDGE,), jnp.int32),
            pltpu.VMEM((W_EDGE,), jnp.int32),
            pltpu.VMEM((Hh, W_EDGE), jnp.float32),
            pltpu.VMEM((W_EDGE,), jnp.int32),
            pltpu.VMEM((16, F), jnp.float32),
            pltpu.VMEM((16, F), jnp.float32),
            pltpu.VMEM((16, 16), jnp.float32),
            pltpu.VMEM((16, F), jnp.float32),
            pltpu.VMEM((16, 16), jnp.float32),
            pltpu.VMEM_SHARED((CHUNK + 16, F), jnp.float32),
            pltpu.VMEM_SHARED((CHUNK + 16, 16), jnp.float32),
            pltpu.VMEM((16,), jnp.int32),
            pltpu.VMEM((16,), jnp.int32),
            pltpu.SemaphoreType.DMA,
            pltpu.SemaphoreType.DMA,
            pltpu.SemaphoreType.DMA,
        ],
    )
    def seg_agg(src_hbm, dst_hbm, e_hbm, feat_hbm, acc_hbm, dacc_hbm,
                src_w, dst_w, e_w, eloc_c, gbuf, msg, dmsg, zbuf, dzbuf,
                acc_sp, dacc_sp, gidx, sidx, gsem, msem, dsem):
        cid = lax.axis_index("c")
        sid = lax.axis_index("s")
        base = sid * (2 * S_EDGE)
        lanes = lax.iota(jnp.int32, 16)
        zeros16 = jnp.zeros((16,), jnp.float32)

        for i in range(16):
            dmsg[i, :] = zeros16
            dzbuf[i, :] = zeros16

            def zrow(j, c0):
                zbuf[i, pl.ds(j * 16, 16)] = zeros16
                return c0

            lax.fori_loop(0, F // 16, zrow, 0)

        def pass_body(p, c1):
            chunk = p * 2 + cid
            lo = chunk * CHUNK
            rows0 = sid * RPT
            for r in range(RPT // 8):
                pltpu.sync_copy(zbuf.at[pl.ds(0, 8)],
                                acc_sp.at[pl.ds(rows0 + r * 8, 8)])
                pltpu.sync_copy(dzbuf.at[pl.ds(0, 8)],
                                dacc_sp.at[pl.ds(rows0 + r * 8, 8)])

            @pl.when(sid == 0)
            def _():
                pltpu.sync_copy(zbuf, acc_sp.at[pl.ds(CHUNK, 16)])
                pltpu.sync_copy(dzbuf, dacc_sp.at[pl.ds(CHUNK, 16)])

            plsc.subcore_barrier()

            def win_body(w, c2):
                off = base + w * W_EDGE
                pltpu.sync_copy(src_hbm.at[pl.ds(off, W_EDGE)], src_w)
                pltpu.sync_copy(dst_hbm.at[pl.ds(off, W_EDGE)], dst_w)
                pltpu.sync_copy(e_hbm.at[:, pl.ds(off, W_EDGE)], e_w)

                def scan_grp(g, n_acc):
                    d16 = dst_w[pl.ds(g * 16, 16)]
                    m = (d16 >= lo) & (d16 < lo + CHUNK)
                    ones = jnp.where(m, 1, 0).astype(jnp.int32)
                    posn = n_acc + plsc.cumsum(ones) - 1
                    plsc.store_scatter(eloc_c, [posn], g * 16 + lanes, mask=m)
                    return n_acc + plsc.all_reduce_population_count(m)

                n_vec = lax.fori_loop(0, W_EDGE // 16,
                                      scan_grp, jnp.zeros((16,), jnp.int32))
                n = n_vec[0]
                ngrp = (n + 15) // 16

                def proc_grp(g2, c3):
                    idxv = g2 * 16 + lanes
                    valid = idxv < n
                    eloc = eloc_c[pl.ds(g2 * 16, 16)]
                    eloc = jnp.where(valid, eloc, 0)
                    d16 = plsc.load_gather(dst_w, [eloc])
                    dstloc = jnp.where(valid, d16 - lo, CHUNK)
                    s16 = plsc.load_gather(src_w, [eloc])
                    s16 = jnp.where(valid, s16, 0)
                    gidx[...] = s16
                    sidx[...] = dstloc
                    pltpu.async_copy(feat_hbm.at[gidx], gbuf, gsem).wait()
                    for h in range(Hh):
                        hful = jnp.full((16,), h, jnp.int32)
                        e16 = plsc.load_gather(e_w, [hful, eloc])
                        e16 = jnp.where(valid, e16, 0.0)
                        plsc.store_scatter(dmsg, [lanes, hful], e16)

                        def col_body(cc, c4):
                            cful = jnp.full((16,), h * HID, jnp.int32) + cc
                            col = plsc.load_gather(gbuf, [lanes, cful])
                            plsc.store_scatter(msg, [lanes, cful], col * e16)
                            return c4

                        lax.fori_loop(0, HID, col_body, 0)
                    pltpu.async_copy(msg, acc_sp.at[sidx], msem,
                                     add=True).wait()
                    pltpu.async_copy(dmsg, dacc_sp.at[sidx], dsem,
                                     add=True).wait()
                    return c3

                lax.fori_loop(0, ngrp, proc_grp, 0)
                return c2

            lax.fori_loop(0, 2 * NWIN, win_body, 0)
            plsc.subcore_barrier()

            out0 = lo + sid * RPT
            for r in range(RPT // 8):
                pltpu.sync_copy(acc_sp.at[pl.ds(rows0 + r * 8, 8)],
                                acc_hbm.at[pl.ds(out0 + r * 8, 8)])
                pltpu.sync_copy(dacc_sp.at[pl.ds(rows0 + r * 8, 8)],
                                dacc_hbm.at[pl.ds(out0 + r * 8, 8)])
            plsc.subcore_barrier()
            return c1

        lax.fori_loop(0, NPASS, pass_body, 0)

    return seg_agg


_edge_e8 = _make_edge_e(H1)
_edge_e1 = _make_edge_e(1)
_agg1 = _make_seg_agg(H1, F1, 640, 16)
_agg2 = _make_seg_agg(1, F2, 5120, 2)


def kernel(x, edge_index, W1, att_src1, att_dst1, b1,
           W2, att_src2, att_dst2, b2):
    x_pad = jnp.pad(x, ((0, NPAD - N_NODES), (0, 0)))
    loops = jnp.arange(N_NODES, dtype=jnp.int32)
    pad_e = EPAD - E_SL
    src = jnp.concatenate([edge_index[0].astype(jnp.int32), loops,
                           jnp.zeros((pad_e,), jnp.int32)])
    dst = jnp.concatenate([edge_index[1].astype(jnp.int32), loops,
                           jnp.full((pad_e,), NPAD - 1, jnp.int32)])

    h1, asT1, adT1 = _pre1(x_pad, W1, att_src1, att_dst1)
    m1 = jnp.full((16,), jnp.max(asT1) + jnp.max(adT1), jnp.float32)
    eT1 = _edge_e8(src, dst, asT1, adT1, m1)
    acc1, dacc1 = _agg1(src, dst, eT1, h1)

    h2, asT2, adT2 = _mid(acc1, dacc1, b1.reshape(1, F1),
                          W2, att_src2, att_dst2)
    m2 = jnp.full((16,), jnp.max(asT2) + jnp.max(adT2), jnp.float32)
    eT2 = _edge_e1(src, dst, asT2, adT2, m2)
    acc2, dacc2 = _agg2(src, dst, eT2, h2)

    out = _post(acc2, dacc2, b2.reshape(1, F2))
    return out[:N_NODES]
